# baseline (speedup 1.0000x reference)
import functools

import jax
import jax.numpy as jnp
from jax import lax
from jax.experimental import pallas as pl
from jax.experimental.pallas import tpu as pltpu
from jax.experimental.pallas import tpu_sc as plsc

_B, _S, _D = 4, 8192, 1024
_NC, _NS = 2, 16
_NW = _NC * _NS
_ROWS_W = _S // _NW
_R = 16
_NCH = _ROWS_W // _R
_LANES = 16
_VPR = _D // _LANES

_mesh = plsc.VectorSubcoreMesh(core_axis_name="c", subcore_axis_name="s")


@functools.partial(
    pl.kernel,
    mesh=_mesh,
    out_type=jax.ShapeDtypeStruct((_B, _S, _D), jnp.float32),
    scratch_types=[
        pltpu.VMEM((_R, _D), jnp.float32),
        pltpu.VMEM((_R, _D), jnp.float32),
        pltpu.VMEM((_R, _D), jnp.float32),
        pltpu.SemaphoreType.DMA,
        pltpu.SemaphoreType.DMA,
        pltpu.SemaphoreType.DMA,
        pltpu.SemaphoreType.DMA,
    ],
)
def _sc_add(x_hbm, pos_hbm, out_hbm, pos_v, xv0, xv1, in0, in1, ou0, ou1):
    wid = lax.axis_index("s") * _NC + lax.axis_index("c")
    base = wid * _ROWS_W
    xv = (xv0, xv1)
    in_sem = (in0, in1)
    out_sem = (ou0, ou1)

    def add_chunk(buf):
        def row_body(r, carry):
            for j in range(_VPR):
                sl = pl.ds(j * _LANES, _LANES)
                buf[r, sl] = buf[r, sl] + pos_v[r, sl]
            return carry

        lax.fori_loop(0, _R, row_body, 0)

    def chunk_body(c, carry):
        row0 = base + c * _R
        sl_rows = pl.ds(row0, _R)
        pltpu.sync_copy(pos_hbm.at[sl_rows], pos_v)

        in_h = [None] * _B
        out_h = [None] * _B
        in_h[0] = pltpu.async_copy(x_hbm.at[0, sl_rows], xv[0], in_sem[0])
        for b in range(_B):
            if b + 1 < _B:
                if b >= 1:
                    out_h[b - 1].wait()
                in_h[b + 1] = pltpu.async_copy(
                    x_hbm.at[b + 1, sl_rows], xv[(b + 1) % 2], in_sem[(b + 1) % 2]
                )
            in_h[b].wait()
            add_chunk(xv[b % 2])
            out_h[b] = pltpu.async_copy(
                xv[b % 2], out_hbm.at[b, sl_rows], out_sem[b % 2]
            )
        out_h[_B - 2].wait()
        out_h[_B - 1].wait()
        return carry

    lax.fori_loop(0, _NCH, chunk_body, 0)


def kernel(x, pos_embedding):
    return _sc_add(x, pos_embedding)


# speedup vs baseline: 1.0457x; 1.0457x over previous
import functools

import jax
import jax.numpy as jnp
from jax import lax
from jax.experimental import pallas as pl
from jax.experimental.pallas import tpu as pltpu
from jax.experimental.pallas import tpu_sc as plsc

_B, _S, _D = 4, 8192, 1024
_NC, _NS = 2, 16
_NW = _NC * _NS
_ROWS_W = _S // _NW
_R = 32
_NCH = _ROWS_W // _R
_LANES = 16
_VPR = _D // _LANES

_mesh = plsc.VectorSubcoreMesh(core_axis_name="c", subcore_axis_name="s")


@functools.partial(
    pl.kernel,
    mesh=_mesh,
    out_type=jax.ShapeDtypeStruct((_B, _S, _D), jnp.float32),
    scratch_types=[
        pltpu.VMEM((_R, _D), jnp.float32),
        pltpu.VMEM((_R, _D), jnp.float32),
        pltpu.VMEM((_R, _D), jnp.float32),
        pltpu.SemaphoreType.DMA,
        pltpu.SemaphoreType.DMA,
        pltpu.SemaphoreType.DMA,
        pltpu.SemaphoreType.DMA,
    ],
)
def _sc_add(x_hbm, pos_hbm, out_hbm, pos_v, xv0, xv1, in0, in1, ou0, ou1):
    wid = lax.axis_index("s") * _NC + lax.axis_index("c")
    base = wid * _ROWS_W
    xv = (xv0, xv1)
    in_sem = (in0, in1)
    out_sem = (ou0, ou1)

    def add_chunk(buf):
        def row_body(r, carry):
            for j in range(_VPR):
                sl = pl.ds(j * _LANES, _LANES)
                buf[r, sl] = buf[r, sl] + pos_v[r, sl]
            return carry

        lax.fori_loop(0, _R, row_body, 0)

    def chunk_body(c, carry):
        row0 = base + c * _R
        sl_rows = pl.ds(row0, _R)
        pltpu.sync_copy(pos_hbm.at[sl_rows], pos_v)

        in_h = [None] * _B
        out_h = [None] * _B
        in_h[0] = pltpu.async_copy(x_hbm.at[0, sl_rows], xv[0], in_sem[0])
        for b in range(_B):
            if b + 1 < _B:
                if b >= 1:
                    out_h[b - 1].wait()
                in_h[b + 1] = pltpu.async_copy(
                    x_hbm.at[b + 1, sl_rows], xv[(b + 1) % 2], in_sem[(b + 1) % 2]
                )
            in_h[b].wait()
            add_chunk(xv[b % 2])
            out_h[b] = pltpu.async_copy(
                xv[b % 2], out_hbm.at[b, sl_rows], out_sem[b % 2]
            )
        out_h[_B - 2].wait()
        out_h[_B - 1].wait()
        return carry

    lax.fori_loop(0, _NCH, chunk_body, 0)


def kernel(x, pos_embedding):
    return _sc_add(x, pos_embedding)
